# baseline (speedup 1.0000x reference)
import jax
import jax.numpy as jnp
from jax import lax
from jax.experimental import pallas as pl
from jax.experimental.pallas import tpu as pltpu
from jax.experimental.pallas import tpu_sc as plsc

N = 10000
E = 160000
D_IN = 128
D_E = 16
D = 32
D_EH = 128
STEPS = 6

N_PAD = 10240
E_PAD = 163840
IDX_ROWS = E_PAD // 128
ROWS_PER_TILE = IDX_ROWS // 32
HALF = ROWS_PER_TILE // 2

_mesh = plsc.VectorSubcoreMesh(core_axis_name="c", subcore_axis_name="s")
_sc_params = pltpu.CompilerParams(use_tc_tiling_on_sc=False)


def _lin0_body(x_ref, w_ref, b_ref, o_ref):
    acc = jnp.dot(x_ref[...], w_ref[...], preferred_element_type=jnp.float32)
    o_ref[...] = jnp.maximum(acc + b_ref[0], 0.0)


def _lin0(x, w, b):
    return pl.pallas_call(
        _lin0_body,
        grid=(N_PAD // 1024,),
        in_specs=[
            pl.BlockSpec((1024, D_IN), lambda i: (i, 0)),
            pl.BlockSpec((D_IN, D), lambda i: (0, 0)),
            pl.BlockSpec((1, D), lambda i: (0, 0)),
        ],
        out_specs=pl.BlockSpec((1024, D), lambda i: (i, 0)),
        out_shape=jax.ShapeDtypeStruct((N_PAD, D), jnp.float32),
    )(x, w, b)


def _wegen_body(e_ref, w1_ref, b1_ref, w2_ref, b2_ref, o_ref):
    a = jnp.dot(e_ref[...], w1_ref[...], preferred_element_type=jnp.float32)
    a = jnp.maximum(a + b1_ref[0], 0.0)
    a = a.astype(jnp.bfloat16).astype(jnp.float32)
    we = jnp.dot(a, w2_ref[...], preferred_element_type=jnp.float32) + b2_ref[0]
    o_ref[...] = we.astype(jnp.bfloat16)


def _wegen(e_feat_p, w1, b1, w2, b2):
    blk = 2048
    return pl.pallas_call(
        _wegen_body,
        grid=(E_PAD // blk,),
        in_specs=[
            pl.BlockSpec((blk, D_E), lambda i: (i, 0)),
            pl.BlockSpec((D_E, D_EH), lambda i: (0, 0)),
            pl.BlockSpec((1, D_EH), lambda i: (0, 0)),
            pl.BlockSpec((D_EH, D * D), lambda i: (0, 0)),
            pl.BlockSpec((1, D * D), lambda i: (0, 0)),
        ],
        out_specs=pl.BlockSpec((blk, D * D), lambda i: (i, 0)),
        out_shape=jax.ShapeDtypeStruct((E_PAD, D * D), jnp.bfloat16),
    )(e_feat_p, w1, b1, w2, b2)


def _msg_body(we_ref, hs_ref, o_ref):
    h = hs_ref[...].astype(jnp.bfloat16).astype(jnp.float32)
    w = we_ref[...]
    acc = h[:, 0:1] * w[:, 0:D].astype(jnp.float32)
    for d in range(1, D):
        acc = acc + h[:, d:d + 1] * w[:, d * D:(d + 1) * D].astype(jnp.float32)
    o_ref[...] = acc


def _msg(we, h_src):
    blk = 2048
    return pl.pallas_call(
        _msg_body,
        grid=(E_PAD // blk,),
        in_specs=[
            pl.BlockSpec((blk, D * D), lambda i: (i, 0)),
            pl.BlockSpec((blk, D), lambda i: (i, 0)),
        ],
        out_specs=pl.BlockSpec((blk, D), lambda i: (i, 0)),
        out_shape=jax.ShapeDtypeStruct((E_PAD, D), jnp.float32),
    )(we, h_src)


def _gru_body(p_ref, h_ref, cb_ref, wih_ref, whh_ref, bih_ref, bhh_ref, o_ref):
    h = h_ref[...]
    agg = p_ref[0] + p_ref[1] + cb_ref[0]
    agg = agg.astype(jnp.bfloat16).astype(jnp.float32)
    m = jnp.maximum(agg, 0.0)
    gi = jnp.dot(m, wih_ref[...], preferred_element_type=jnp.float32) + bih_ref[0]
    gh = jnp.dot(h, whh_ref[...], preferred_element_type=jnp.float32) + bhh_ref[0]
    r = jax.nn.sigmoid(gi[:, :D] + gh[:, :D])
    z = jax.nn.sigmoid(gi[:, D:2 * D] + gh[:, D:2 * D])
    n = jnp.tanh(gi[:, 2 * D:] + r * gh[:, 2 * D:])
    o_ref[...] = (1.0 - z) * n + z * h


def _gru(partials, h, cb, wih_t, whh_t, bih, bhh):
    blk = 1024
    return pl.pallas_call(
        _gru_body,
        grid=(N_PAD // blk,),
        in_specs=[
            pl.BlockSpec((2, blk, D), lambda i: (0, i, 0)),
            pl.BlockSpec((blk, D), lambda i: (i, 0)),
            pl.BlockSpec((1, D), lambda i: (0, 0)),
            pl.BlockSpec((D, 3 * D), lambda i: (0, 0)),
            pl.BlockSpec((D, 3 * D), lambda i: (0, 0)),
            pl.BlockSpec((1, 3 * D), lambda i: (0, 0)),
            pl.BlockSpec((1, 3 * D), lambda i: (0, 0)),
        ],
        out_specs=pl.BlockSpec((blk, D), lambda i: (i, 0)),
        out_shape=jax.ShapeDtypeStruct((N_PAD, D), jnp.float32),
    )(partials, h, cb, wih_t, whh_t, bih, bhh)


def _gather_tec(h_hbm, srcm_hbm, out_hbm, idx_v, rows_v, sem):
    wid = lax.axis_index("s") * 2 + lax.axis_index("c")
    base = wid * ROWS_PER_TILE
    pltpu.sync_copy(srcm_hbm.at[pl.ds(base, ROWS_PER_TILE)], idx_v)
    for c in range(2):
        cps = [
            pltpu.async_copy(h_hbm.at[idx_v.at[c * HALF + j]], rows_v.at[j], sem)
            for j in range(HALF)
        ]
        for cp in cps:
            cp.wait()
        pltpu.sync_copy(rows_v, out_hbm.at[pl.ds(base + c * HALF, HALF)])


def _gather(h, srcm):
    k = pl.kernel(
        _gather_tec,
        out_type=jax.ShapeDtypeStruct((IDX_ROWS, 128, D), jnp.float32),
        mesh=_mesh,
        scratch_types=[
            pltpu.VMEM((ROWS_PER_TILE, 128), jnp.int32),
            pltpu.VMEM((HALF, 128, D), jnp.float32),
            pltpu.SemaphoreType.DMA,
        ],
        compiler_params=_sc_params,
    )
    return k(h, srcm)


def _scatter_tec(msg_hbm, dstm_hbm, zeros_hbm, out_hbm, idx_v, rows_v, shared, sem):
    c_ax = lax.axis_index("c")
    s_ax = lax.axis_index("s")
    wid = s_ax * 2 + c_ax

    @pl.when(s_ax == 0)
    def _():
        pltpu.sync_copy(zeros_hbm, shared)

    plsc.subcore_barrier()
    base = wid * ROWS_PER_TILE
    pltpu.sync_copy(dstm_hbm.at[pl.ds(base, ROWS_PER_TILE)], idx_v)
    for c in range(2):
        pltpu.sync_copy(msg_hbm.at[pl.ds(base + c * HALF, HALF)], rows_v)
        for j in range(HALF):
            pltpu.sync_copy(rows_v.at[j], shared.at[idx_v.at[c * HALF + j]], add=True)
    plsc.subcore_barrier()
    nrows = N_PAD // 16
    pltpu.sync_copy(shared.at[pl.ds(s_ax * nrows, nrows)],
                    out_hbm.at[c_ax, pl.ds(s_ax * nrows, nrows)])


def _scatter(msg, dstm, zeros_np):
    k = pl.kernel(
        _scatter_tec,
        out_type=jax.ShapeDtypeStruct((2, N_PAD, D), jnp.float32),
        mesh=_mesh,
        scratch_types=[
            pltpu.VMEM((ROWS_PER_TILE, 128), jnp.int32),
            pltpu.VMEM((HALF, 128, D), jnp.float32),
            pltpu.VMEM_SHARED((N_PAD, D), jnp.float32),
            pltpu.SemaphoreType.DMA,
        ],
        compiler_params=_sc_params,
    )
    return k(msg, dstm, zeros_np)


def kernel(n_feat, edge_index, e_feat, lin0_W, lin0_b, en_W1, en_b1, en_W2,
           en_b2, conv_bias, gru_Wih, gru_Whh, gru_bih, gru_bhh):
    f32 = jnp.float32
    src = edge_index[0]
    dst = edge_index[1]
    srcm = jnp.pad(src, (0, E_PAD - E)).reshape(IDX_ROWS, 128)
    dstm = jnp.pad(dst, (0, E_PAD - E), constant_values=N).reshape(IDX_ROWS, 128)
    e_feat_p = jnp.pad(e_feat, ((0, E_PAD - E), (0, 0)))
    n_feat_p = jnp.pad(n_feat, ((0, N_PAD - N), (0, 0)))
    zeros_np = jnp.zeros((N_PAD, D), f32)

    h = _lin0(n_feat_p, lin0_W, lin0_b.reshape(1, D))
    we = _wegen(e_feat_p, en_W1, en_b1.reshape(1, D_EH),
                en_W2, en_b2.reshape(1, D * D))

    cb = conv_bias.reshape(1, D)
    wih_t = gru_Wih.T
    whh_t = gru_Whh.T
    bih = gru_bih.reshape(1, 3 * D)
    bhh = gru_bhh.reshape(1, 3 * D)

    for _ in range(STEPS):
        h_src = _gather(h, srcm)
        msg = _msg(we, h_src.reshape(E_PAD, D))
        partials = _scatter(msg.reshape(IDX_ROWS, 128, D), dstm, zeros_np)
        h = _gru(partials, h, cb, wih_t, whh_t, bih, bhh)

    return h[:N]
